# baseline (speedup 1.0000x reference)
import functools

import jax
import jax.numpy as jnp
from jax import lax
from jax.experimental import pallas as pl
from jax.experimental.pallas import tpu as pltpu
from jax.experimental.pallas import tpu_sc as plsc

TM = 5000
NS = 4
KD = 5
NCAND = NS * KD * 8
SPLIT = 16


def _tc_body(obs_ref, W_ref, bT_ref, mem_ref, out_ref, embT_ref, q2_ref,
             run_ref, *, n_tiles):
    i = pl.program_id(0)

    @pl.when(i == 0)
    def _init():
        embT = jax.lax.dot_general(
            W_ref[...], obs_ref[...], (((0,), (1,)), ((), ())),
            preferred_element_type=jnp.float32)
        embT = embT + bT_ref[...]
        q2 = jnp.sum(embT * embT, axis=0, keepdims=True)
        q2_ref[...] = jnp.broadcast_to(q2, q2_ref.shape)
        embT_ref[...] = -2.0 * embT
        run_ref[...] = jnp.full(run_ref.shape, jnp.inf, jnp.float32)

    mem = mem_ref[...]
    qm = jax.lax.dot_general(
        mem, embT_ref[...], (((1,), (0,)), ((), ())),
        preferred_element_type=jnp.float32)
    m2 = jnp.sum(mem * mem, axis=1, keepdims=True)
    s = qm + m2

    stacks = [[run_ref[(st * KD + j) * 8:(st * KD + j) * 8 + 8, :]
               for j in range(KD)] for st in range(NS)]
    for r in range(TM // 8):
        t = s[r * 8:r * 8 + 8, :]
        b = stacks[r % NS]
        for j in range(KD):
            lo = jnp.minimum(b[j], t)
            t = jnp.maximum(b[j], t)
            b[j] = lo
    for st in range(NS):
        for j in range(KD):
            base = (st * KD + j) * 8
            run_ref[base:base + 8, :] = stacks[st][j]

    @pl.when(i == n_tiles - 1)
    def _fin():
        cand = jnp.concatenate([stacks[st][j] for st in range(NS)
                                for j in range(KD)], axis=0)
        cand = cand + q2_ref[0:1, :]
        out_ref[...] = lax.transpose(cand, (1, 0))


def _tc_candidates(obs, memory, W, bT, off, n_tiles):
    d_dim = memory.shape[1]
    n_q = obs.shape[0]
    return pl.pallas_call(
        functools.partial(_tc_body, n_tiles=n_tiles),
        grid=(n_tiles,),
        in_specs=[
            pl.BlockSpec(obs.shape, lambda i: (0, 0)),
            pl.BlockSpec(W.shape, lambda i: (0, 0)),
            pl.BlockSpec((d_dim, 1), lambda i: (0, 0)),
            pl.BlockSpec((TM, d_dim), lambda i: (i + off, 0)),
        ],
        out_specs=pl.BlockSpec((n_q, NCAND), lambda i: (0, 0)),
        out_shape=jax.ShapeDtypeStruct((n_q, NCAND), jnp.float32),
        scratch_shapes=[
            pltpu.VMEM((d_dim, n_q), jnp.float32),
            pltpu.VMEM((8, n_q), jnp.float32),
            pltpu.VMEM((NCAND, n_q), jnp.float32),
        ],
        compiler_params=pltpu.CompilerParams(
            dimension_semantics=("arbitrary",)),
    )(obs, W, bT, memory)


def _merge16(a, b):
    return jnp.sort(jnp.minimum(a, lax.rev(b, (0,))))


def _fold_cands(cand_v, keep):
    for i in range(NCAND // 16):
        keep = _merge16(keep, jnp.sort(cand_v[pl.ds(i * 16, 16)]))
    return keep


def _sc_body1(cand_hbm, out_hbm, cand_v, out_v, *, n_q):
    info = plsc.get_sparse_core_info()
    wid = lax.axis_index("s") * info.num_cores + lax.axis_index("c")

    @pl.when(wid < n_q)
    def _work():
        pltpu.sync_copy(cand_hbm.at[wid], cand_v)
        keep = jnp.sort(cand_v[pl.ds(0, 16)])
        keep = _fold_cands(cand_v, keep)
        out_v[...] = keep
        pltpu.sync_copy(out_v, out_hbm.at[wid])


def _sc_body2(cand_hbm, keep_hbm, out_hbm, cand_v, keep_v, out_v,
              *, n_q, k_top):
    info = plsc.get_sparse_core_info()
    wid = lax.axis_index("s") * info.num_cores + lax.axis_index("c")

    @pl.when(wid < n_q)
    def _work():
        pltpu.sync_copy(cand_hbm.at[wid], cand_v)
        pltpu.sync_copy(keep_hbm.at[wid], keep_v)
        keep = _fold_cands(cand_v, keep_v[...])
        lane = lax.broadcasted_iota(jnp.int32, (16,), 0)
        mask = lane < k_top
        x = jnp.maximum(jnp.where(mask, keep, 1.0), 0.0) + 1e-12
        xi = lax.bitcast_convert_type(x, jnp.int32)
        y = lax.bitcast_convert_type(
            jnp.int32(0x5F3759DF) - lax.shift_right_arithmetic(xi, 1),
            jnp.float32)
        for _ in range(3):
            y = y * (1.5 - 0.5 * x * y * y)
        r = x * y
        r = 0.5 * (r + x / r)
        out_v[...] = jnp.where(mask, r, 0.0)
        pltpu.sync_copy(out_v, out_hbm.at[wid])


def _sc_merge1(candT, n_q):
    mesh = plsc.VectorSubcoreMesh(core_axis_name="c", subcore_axis_name="s")
    return pl.kernel(
        functools.partial(_sc_body1, n_q=n_q),
        out_type=jax.ShapeDtypeStruct((n_q, 16), jnp.float32),
        mesh=mesh,
        scratch_types=[
            pltpu.VMEM((NCAND,), jnp.float32),
            pltpu.VMEM((16,), jnp.float32),
        ],
        compiler_params=pltpu.CompilerParams(needs_layout_passes=False),
    )(candT)


def _sc_merge2(candT, keep, n_q, k_top):
    mesh = plsc.VectorSubcoreMesh(core_axis_name="c", subcore_axis_name="s")
    return pl.kernel(
        functools.partial(_sc_body2, n_q=n_q, k_top=k_top),
        out_type=jax.ShapeDtypeStruct((n_q, 16), jnp.float32),
        mesh=mesh,
        scratch_types=[
            pltpu.VMEM((NCAND,), jnp.float32),
            pltpu.VMEM((16,), jnp.float32),
            pltpu.VMEM((16,), jnp.float32),
        ],
        compiler_params=pltpu.CompilerParams(needs_layout_passes=False),
    )(candT, keep)


@jax.jit
def _novelty(obs, memory, W, bT):
    n_q = obs.shape[0]
    k_top = 5
    n_tiles = memory.shape[0] // TM
    cand1 = _tc_candidates(obs, memory, W, bT, 0, SPLIT)
    cand2 = _tc_candidates(obs, memory, W, bT, SPLIT, n_tiles - SPLIT)
    keep1 = _sc_merge1(cand1, n_q)
    dists = _sc_merge2(cand2, keep1, n_q, k_top)
    return jnp.sum(dists) / (k_top * n_q)


def kernel(obs, memory, W, b):
    return _novelty(obs, memory, W, b.reshape(-1, 1))


# speedup vs baseline: 1.0538x; 1.0538x over previous
import functools

import jax
import jax.numpy as jnp
from jax import lax
from jax.experimental import pallas as pl
from jax.experimental.pallas import tpu as pltpu
from jax.experimental.pallas import tpu_sc as plsc

TM = 5000
NS = 4
KD = 5
NCAND = NS * KD * 8


def _tc_body(obs_ref, W_ref, bT_ref, mem_ref, out_ref, embT_ref, q2_ref,
             run_ref, *, n_tiles):
    i = pl.program_id(0)

    @pl.when(i == 0)
    def _init():
        embT = jax.lax.dot_general(
            W_ref[...], obs_ref[...], (((0,), (1,)), ((), ())),
            preferred_element_type=jnp.float32)
        embT = embT + bT_ref[...]
        q2 = jnp.sum(embT * embT, axis=0, keepdims=True)
        q2_ref[...] = jnp.broadcast_to(q2, q2_ref.shape)
        embT_ref[...] = -2.0 * embT
        run_ref[...] = jnp.full(run_ref.shape, jnp.inf, jnp.float32)

    mem = mem_ref[...]
    qm = jax.lax.dot_general(
        mem, embT_ref[...], (((1,), (0,)), ((), ())),
        preferred_element_type=jnp.float32)
    m2 = jnp.sum(mem * mem, axis=1, keepdims=True)
    s = qm + m2

    stacks = [[run_ref[(st * KD + j) * 8:(st * KD + j) * 8 + 8, :]
               for j in range(KD)] for st in range(NS)]
    for r in range(TM // 8):
        t = s[r * 8:r * 8 + 8, :]
        b = stacks[r % NS]
        for j in range(KD):
            lo = jnp.minimum(b[j], t)
            t = jnp.maximum(b[j], t)
            b[j] = lo
    for st in range(NS):
        for j in range(KD):
            base = (st * KD + j) * 8
            run_ref[base:base + 8, :] = stacks[st][j]

    @pl.when(i == n_tiles - 1)
    def _fin():
        cand = jnp.concatenate([stacks[st][j] for st in range(NS)
                                for j in range(KD)], axis=0)
        cand = cand + q2_ref[0:1, :]
        out_ref[...] = lax.transpose(cand, (1, 0))


@jax.jit
def _tc_candidates(obs, memory, W, bT):
    m_total, d_dim = memory.shape
    n_q = obs.shape[0]
    n_tiles = m_total // TM
    return pl.pallas_call(
        functools.partial(_tc_body, n_tiles=n_tiles),
        grid=(n_tiles,),
        in_specs=[
            pl.BlockSpec(obs.shape, lambda i: (0, 0)),
            pl.BlockSpec(W.shape, lambda i: (0, 0)),
            pl.BlockSpec((d_dim, 1), lambda i: (0, 0)),
            pl.BlockSpec((TM, d_dim), lambda i: (i, 0)),
        ],
        out_specs=pl.BlockSpec((n_q, NCAND), lambda i: (0, 0)),
        out_shape=jax.ShapeDtypeStruct((n_q, NCAND), jnp.float32),
        scratch_shapes=[
            pltpu.VMEM((d_dim, n_q), jnp.float32),
            pltpu.VMEM((8, n_q), jnp.float32),
            pltpu.VMEM((NCAND, n_q), jnp.float32),
        ],
        compiler_params=pltpu.CompilerParams(
            dimension_semantics=("arbitrary",)),
    )(obs, W, bT, memory)


def _merge16(a, b):
    return jnp.sort(jnp.minimum(a, lax.rev(b, (0,))))


def _sc_body(cand_hbm, out_hbm, cand_v, out_v, *, n_q, k_top):
    info = plsc.get_sparse_core_info()
    nc = info.num_cores
    wid = lax.axis_index("s") * nc + lax.axis_index("c")

    @pl.when(wid < n_q)
    def _work():
        pltpu.sync_copy(cand_hbm.at[wid], cand_v)
        keep = jnp.sort(cand_v[pl.ds(0, 16)])
        for i in range(1, NCAND // 16):
            keep = _merge16(keep, jnp.sort(cand_v[pl.ds(i * 16, 16)]))
        lane = lax.broadcasted_iota(jnp.int32, (16,), 0)
        mask = lane < k_top
        x = jnp.maximum(jnp.where(mask, keep, 1.0), 0.0) + 1e-12
        xi = lax.bitcast_convert_type(x, jnp.int32)
        y = lax.bitcast_convert_type(
            jnp.int32(0x5F3759DF) - lax.shift_right_arithmetic(xi, 1),
            jnp.float32)
        for _ in range(3):
            y = y * (1.5 - 0.5 * x * y * y)
        r = x * y
        r = 0.5 * (r + x / r)
        out_v[...] = jnp.where(mask, r, 0.0)
        pltpu.sync_copy(out_v, out_hbm.at[wid])


def _sc_merge(candT, n_q, k_top):
    mesh = plsc.VectorSubcoreMesh(core_axis_name="c", subcore_axis_name="s")
    return pl.kernel(
        functools.partial(_sc_body, n_q=n_q, k_top=k_top),
        out_type=jax.ShapeDtypeStruct((n_q, 16), jnp.float32),
        mesh=mesh,
        scratch_types=[
            pltpu.VMEM((NCAND,), jnp.float32),
            pltpu.VMEM((16,), jnp.float32),
        ],
        compiler_params=pltpu.CompilerParams(needs_layout_passes=False),
    )(candT)


def kernel(obs, memory, W, b):
    n_q = obs.shape[0]
    k_top = 5
    candT = _tc_candidates(obs, memory, W, b.reshape(-1, 1))
    dists = _sc_merge(candT, n_q, k_top)
    return jnp.sum(dists) / (k_top * n_q)
